# Initial kernel scaffold; baseline (speedup 1.0000x reference)
import jax
import jax.numpy as jnp
from jax.experimental import pallas as pl


def kernel(inputs, token_table, pos_table):
    raise NotImplementedError("write your pallas kernel here")


# speedup vs baseline: 2.9776x; 2.9776x over previous
import functools

import jax
import jax.numpy as jnp
from jax import lax
from jax.experimental import pallas as pl
from jax.experimental.pallas import tpu as pltpu
from jax.experimental.pallas import tpu_sc as plsc

VOCAB = 100000
EMBED = 64
SEQ = 200
BATCH = 4096

NC, NS = 2, 16
NW = NC * NS
ROWS = BATCH * SEQ
ROWS_W = ROWS // NW
CHUNK = 2 * SEQ
SUB = 100
NSUB = CHUNK // SUB
NITER = ROWS_W // CHUNK
IDXROWS = ROWS // SUB
IDXROWS_W = ROWS_W // SUB


def _body(idx_hbm, tok_hbm, pos_hbm, out_hbm, idx_v, pos_v, buf_v, sem):
    wid = lax.axis_index("s") * NC + lax.axis_index("c")
    idx_base = wid * IDXROWS_W
    row_base = wid * ROWS_W

    pltpu.sync_copy(idx_hbm.at[pl.ds(idx_base, IDXROWS_W)], idx_v)
    pltpu.sync_copy(pos_hbm, pos_v)

    def iteration(i, _):
        descs = [
            pltpu.async_copy(
                tok_hbm.at[idx_v.at[i * NSUB + r]],
                buf_v.at[pl.ds(r * SUB, SUB)],
                sem,
            )
            for r in range(NSUB)
        ]
        for d in descs:
            d.wait()

        def add_row(j, _):
            for c in range(EMBED // 16):
                sl = pl.ds(c * 16, 16)
                pv = pos_v[j, sl]
                buf_v[j, sl] += pv
                buf_v[j + SEQ, sl] += pv
            return ()

        lax.fori_loop(0, SEQ, add_row, (), unroll=2)

        pltpu.sync_copy(buf_v, out_hbm.at[pl.ds(row_base + i * CHUNK, CHUNK)])
        return ()

    lax.fori_loop(0, NITER, iteration, ())


@jax.jit
def _embed(idx2d, token_table, pos_table):
    mesh = plsc.VectorSubcoreMesh(core_axis_name="c", subcore_axis_name="s")
    return pl.kernel(
        _body,
        out_type=jax.ShapeDtypeStruct((ROWS, EMBED), jnp.float32),
        mesh=mesh,
        compiler_params=pltpu.CompilerParams(use_tc_tiling_on_sc=False),
        scratch_types=[
            pltpu.VMEM((IDXROWS_W, SUB), jnp.int32),
            pltpu.VMEM((SEQ, EMBED), jnp.float32),
            pltpu.VMEM((CHUNK, EMBED), jnp.float32),
            pltpu.SemaphoreType.DMA,
        ],
    )(idx2d, token_table, pos_table)


def kernel(inputs, token_table, pos_table):
    idx2d = inputs.reshape(IDXROWS, SUB).astype(jnp.int32)
    out = _embed(idx2d, token_table, pos_table)
    return out.reshape(BATCH, SEQ, EMBED)
